# baseline (speedup 1.0000x reference)
import jax
import jax.numpy as jnp
from jax import lax
from jax.experimental import pallas as pl
from jax.experimental.pallas import tpu as pltpu
from jax.experimental.pallas import tpu_sc as plsc

B = 512
SIZE = 128
D = 1024
LEVELS = 256

NC = 2
NS = 16
L = 16
NW = NC * NS
NBB = 4
NDB = NW // NBB
BW = B // NBB
DW = D // NDB
NB = 4
NG = BW // NB
NBUF = 4
SCHUNK = 8

_NIB_ONES = 0x11111111
_NIB_LO = 0x0F0F0F0F


def _quantize(xv):
    y = xv * jnp.float32(LEVELS - 1)
    r = (y + jnp.float32(8388608.0)) - jnp.float32(8388608.0)
    return jnp.clip(r.astype(jnp.int32), 0, LEVELS - 1)


def _minus_bit(fv):
    return lax.shift_right_arithmetic(fv.astype(jnp.int32), jnp.int32(1)) & jnp.int32(1)


def _body(x_hbm, pos_hbm, lev_hbm, out_hbm,
          x_v, idx_v, posp_v, levp_v, out_v, levp_hbm, bufs, sems):
    cid = lax.axis_index("c")
    sid = lax.axis_index("s")
    wid = sid * NC + cid
    row0 = pl.multiple_of((wid % NBB) * BW, BW)
    col0 = pl.multiple_of((wid // NBB) * DW, DW)

    pltpu.sync_copy(x_hbm.at[pl.ds(row0, BW)], x_v)

    def q_loop(b, _):
        for c in range(SIZE // L):
            xv = x_v[b, pl.ds(c * L, L)]
            idx_v[pl.ds(b * SIZE + c * L, L)] = _quantize(xv) + sid * LEVELS
        return _

    lax.fori_loop(0, BW, q_loop, None)

    def _pack_block(dst_v, dst_row0, nrows):
        def pack_loop(r, _):
            w = jnp.zeros((L,), jnp.int32)
            for j in range(DW // L):
                fv = out_v[r, pl.ds(j * L, L)]
                w = w | lax.shift_left(_minus_bit(fv), jnp.int32(4 * j))
            dst_v[dst_row0 + r] = w
            return _

        lax.fori_loop(0, nrows, pack_loop, None)

    pltpu.sync_copy(pos_hbm.at[:, pl.ds(col0, DW)], out_v)
    _pack_block(posp_v, 0, SIZE)
    pltpu.sync_copy(lev_hbm.at[pl.ds(0, BW), pl.ds(col0, DW)], out_v)
    _pack_block(levp_v, 0, BW)
    pltpu.sync_copy(lev_hbm.at[pl.ds(BW, BW), pl.ds(col0, DW)], out_v)
    _pack_block(levp_v, BW, BW)

    pltpu.sync_copy(levp_v, levp_hbm.at[pl.ds(sid * LEVELS, LEVELS)])
    plsc.subcore_barrier()

    nib_ones = jnp.full((L,), _NIB_ONES, jnp.int32)
    nib_lo = jnp.full((L,), _NIB_LO, jnp.int32)

    def _start_gather(g, k):
        for nb in range(NB):
            pltpu.async_copy(
                levp_hbm.at[idx_v.at[pl.ds((g * NB + nb) * SIZE, SIZE)]],
                bufs[k].at[pl.ds(nb * SIZE, SIZE)],
                sems[k],
            )

    def _wait_gather(k):
        for nb in range(NB):
            pltpu.make_async_copy(
                levp_hbm.at[idx_v.at[pl.ds(nb * SIZE, SIZE)]],
                bufs[k].at[pl.ds(nb * SIZE, SIZE)],
                sems[k],
            ).wait()

    for k in range(NBUF):
        _start_gather(k, k)

    def g_loop(j, _):
        for k in range(NBUF):
            g = j * NBUF + k
            _wait_gather(k)
            buf = bufs[k]

            def s8_loop(t, acc8):
                acc8lo, acc8hi = acc8
                acc4 = [jnp.zeros((L,), jnp.int32) for _ in range(NB)]
                for u in range(SCHUNK):
                    s = t * SCHUNK + u
                    pw = posp_v[s]
                    for nb in range(NB):
                        lw = buf[nb * SIZE + s]
                        acc4[nb] = acc4[nb] + ((lw ^ pw) & nib_ones)
                acc8lo = [acc8lo[nb] + (acc4[nb] & nib_lo) for nb in range(NB)]
                acc8hi = [
                    acc8hi[nb]
                    + (lax.shift_right_logical(acc4[nb], jnp.int32(4)) & nib_lo)
                    for nb in range(NB)
                ]
                return acc8lo, acc8hi

            zeros = [jnp.zeros((L,), jnp.int32) for _ in range(NB)]
            acc8lo, acc8hi = lax.fori_loop(
                0, SIZE // SCHUNK, s8_loop, (list(zeros), list(zeros))
            )
            for nb in range(NB):
                for jj in range(4):
                    for half, acc in ((0, acc8lo[nb]), (1, acc8hi[nb])):
                        cnt = lax.shift_right_logical(
                            acc, jnp.int32(8 * jj)
                        ) & jnp.int32(0xFF)
                        val = (
                            jnp.int32(SIZE) - lax.shift_left(cnt, jnp.int32(1))
                        ).astype(jnp.float32)
                        out_v[g * NB + nb, pl.ds(32 * jj + 16 * half, L)] = val
        for k in range(NBUF):
            gn = lax.rem((j + 1) * NBUF + k, jnp.int32(NG))
            _start_gather(gn, k)
        return _

    lax.fori_loop(0, NG // NBUF, g_loop, None)
    for k in range(NBUF):
        _wait_gather(k)
    pltpu.sync_copy(out_v, out_hbm.at[pl.ds(row0, BW), pl.ds(col0, DW)])


@jax.jit
def kernel(x, position, level):
    mesh = plsc.VectorSubcoreMesh(
        core_axis_name="c", subcore_axis_name="s", num_cores=NC, num_subcores=NS
    )

    def body(x_hbm, pos_hbm, lev_hbm, out_hbm, x_v, idx_v, posp_v, levp_v,
             out_v, levp_hbm, b0, b1, b2, b3, s0, s1, s2, s3):
        _body(x_hbm, pos_hbm, lev_hbm, out_hbm, x_v, idx_v, posp_v, levp_v,
              out_v, levp_hbm, [b0, b1, b2, b3], [s0, s1, s2, s3])

    return pl.kernel(
        body,
        out_type=jax.ShapeDtypeStruct((B, D), jnp.float32),
        mesh=mesh,
        compiler_params=pltpu.CompilerParams(
            needs_layout_passes=False, use_tc_tiling_on_sc=False
        ),
        scratch_types=[
            pltpu.VMEM((BW, SIZE), jnp.float32),
            pltpu.VMEM((BW * SIZE,), jnp.int32),
            pltpu.VMEM((SIZE, L), jnp.int32),
            pltpu.VMEM((LEVELS, L), jnp.int32),
            pltpu.VMEM((BW, DW), jnp.float32),
            pltpu.VMEM_SHARED((NS * LEVELS, L), jnp.int32),
            pltpu.VMEM((NB * SIZE, L), jnp.int32),
            pltpu.VMEM((NB * SIZE, L), jnp.int32),
            pltpu.VMEM((NB * SIZE, L), jnp.int32),
            pltpu.VMEM((NB * SIZE, L), jnp.int32),
            pltpu.SemaphoreType.DMA,
            pltpu.SemaphoreType.DMA,
            pltpu.SemaphoreType.DMA,
            pltpu.SemaphoreType.DMA,
        ],
    )(x, position, level)


# speedup vs baseline: 1.0780x; 1.0780x over previous
import jax
import jax.numpy as jnp
from jax import lax
from jax.experimental import pallas as pl
from jax.experimental.pallas import tpu as pltpu
from jax.experimental.pallas import tpu_sc as plsc

B = 512
SIZE = 128
D = 1024
LEVELS = 256

NC = 2
NS = 16
L = 16
NW = NC * NS
NBB = 4
NDB = NW // NBB
BW = B // NBB
DW = D // NDB
NB = 4
SCHUNK = 8

_NIB_ONES = 0x11111111
_NIB_LO = 0x0F0F0F0F


def _quantize(xv):
    y = xv * jnp.float32(LEVELS - 1)
    r = (y + jnp.float32(8388608.0)) - jnp.float32(8388608.0)
    return jnp.clip(r.astype(jnp.int32), 0, LEVELS - 1)


def _minus_bit(fv):
    return lax.shift_right_arithmetic(fv.astype(jnp.int32), jnp.int32(1)) & jnp.int32(1)


def _body(x_hbm, pos_hbm, lev_hbm, out_hbm, x_v, idx_v, posp_v, levp_v, out_v):
    cid = lax.axis_index("c")
    sid = lax.axis_index("s")
    wid = sid * NC + cid
    row0 = pl.multiple_of((wid % NBB) * BW, BW)
    col0 = pl.multiple_of((wid // NBB) * DW, DW)

    pltpu.sync_copy(x_hbm.at[pl.ds(row0, BW)], x_v)

    iota = lax.iota(jnp.int32, L)

    def q_loop(b, _):
        for c in range(SIZE // L):
            xv = x_v[b, pl.ds(c * L, L)]
            idx_v[pl.ds(b * SIZE + c * L, L)] = lax.shift_left(
                _quantize(xv), jnp.int32(4)
            )
        return _

    lax.fori_loop(0, BW, q_loop, None)

    def _pack_block(dst_v, dst_row0, nrows):
        def pack_loop(r, off):
            w = jnp.zeros((L,), jnp.int32)
            for j in range(DW // L):
                fv = out_v[r, pl.ds(j * L, L)]
                w = w | lax.shift_left(_minus_bit(fv), jnp.int32(4 * j))
            dst_v[pl.ds(off, L)] = w
            return off + L

        lax.fori_loop(0, nrows, pack_loop, dst_row0 * L)

    pltpu.sync_copy(pos_hbm.at[:, pl.ds(col0, DW)], out_v)
    _pack_block(posp_v, 0, SIZE)
    pltpu.sync_copy(lev_hbm.at[pl.ds(0, BW), pl.ds(col0, DW)], out_v)
    _pack_block(levp_v, 0, BW)
    pltpu.sync_copy(lev_hbm.at[pl.ds(BW, BW), pl.ds(col0, DW)], out_v)
    _pack_block(levp_v, BW, BW)

    nib_ones = jnp.full((L,), _NIB_ONES, jnp.int32)
    nib_lo = jnp.full((L,), _NIB_LO, jnp.int32)

    def b_loop(b0, _):
        base = b0 * NB * SIZE

        def s8_loop(t, acc8):
            acc8lo, acc8hi = acc8
            acc4 = [jnp.zeros((L,), jnp.int32) for _ in range(NB)]
            for u in range(SCHUNK):
                s = t * SCHUNK + u
                pw = posp_v[pl.ds(s * L, L)]
                bvec = jnp.broadcast_to(base + s, (L,))
                for nb in range(NB):
                    row = plsc.load_gather(idx_v, [bvec + jnp.int32(nb * SIZE)])
                    lw = plsc.load_gather(levp_v, [row + iota])
                    acc4[nb] = acc4[nb] + ((lw ^ pw) & nib_ones)
            acc8lo = [acc8lo[nb] + (acc4[nb] & nib_lo) for nb in range(NB)]
            acc8hi = [
                acc8hi[nb]
                + (lax.shift_right_logical(acc4[nb], jnp.int32(4)) & nib_lo)
                for nb in range(NB)
            ]
            return acc8lo, acc8hi

        zeros = [jnp.zeros((L,), jnp.int32) for _ in range(NB)]
        acc8lo, acc8hi = lax.fori_loop(
            0, SIZE // SCHUNK, s8_loop, (list(zeros), list(zeros))
        )
        for nb in range(NB):
            for j in range(4):
                for half, acc in ((0, acc8lo[nb]), (1, acc8hi[nb])):
                    cnt = lax.shift_right_logical(acc, jnp.int32(8 * j)) & jnp.int32(0xFF)
                    val = (jnp.int32(SIZE) - lax.shift_left(cnt, jnp.int32(1))).astype(
                        jnp.float32
                    )
                    out_v[b0 * NB + nb, pl.ds(32 * j + 16 * half, L)] = val
        return _

    lax.fori_loop(0, BW // NB, b_loop, None)
    pltpu.sync_copy(out_v, out_hbm.at[pl.ds(row0, BW), pl.ds(col0, DW)])


@jax.jit
def kernel(x, position, level):
    mesh = plsc.VectorSubcoreMesh(
        core_axis_name="c", subcore_axis_name="s", num_cores=NC, num_subcores=NS
    )
    return pl.kernel(
        _body,
        out_type=jax.ShapeDtypeStruct((B, D), jnp.float32),
        mesh=mesh,
        compiler_params=pltpu.CompilerParams(needs_layout_passes=False),
        scratch_types=[
            pltpu.VMEM((BW, SIZE), jnp.float32),
            pltpu.VMEM((BW * SIZE,), jnp.int32),
            pltpu.VMEM((SIZE * L,), jnp.int32),
            pltpu.VMEM((LEVELS * L,), jnp.int32),
            pltpu.VMEM((BW, DW), jnp.float32),
        ],
    )(x, position, level)
